# baseline (speedup 1.0000x reference)
import functools

import jax
import jax.numpy as jnp
from jax import lax
from jax.experimental import pallas as pl
from jax.experimental.pallas import tpu as pltpu
from jax.experimental.pallas import tpu_sc as plsc

NUM_CORES = 2
NUM_SUBCORES = 16
NUM_WORKERS = NUM_CORES * NUM_SUBCORES
CHUNK_ROWS = 120


def kernel(tokens, embed_table):
    batch = tokens.shape[0]
    seq_len = tokens.shape[1]
    embed_dim = embed_table.shape[1]
    rows_per_worker = seq_len // NUM_WORKERS
    chunks = []
    off = 0
    while off < rows_per_worker:
        size = min(CHUNK_ROWS, rows_per_worker - off)
        chunks.append((off, size))
        off += size
    mesh = plsc.VectorSubcoreMesh(core_axis_name="c", subcore_axis_name="s")

    @functools.partial(
        pl.kernel,
        mesh=mesh,
        out_type=jax.ShapeDtypeStruct(
            (batch, seq_len, embed_dim), embed_table.dtype),
        scratch_types=[
            pltpu.VMEM((CHUNK_ROWS, embed_dim), jnp.float32),
            pltpu.SemaphoreType.DMA,
        ],
    )
    def sc_copy(table_hbm, out_hbm, buf, wsem):
        wid = lax.axis_index("s") * NUM_CORES + lax.axis_index("c")
        base = wid * rows_per_worker

        for off, size in chunks:
            r = base + off
            pltpu.sync_copy(table_hbm.at[pl.ds(r, size)],
                            buf.at[pl.ds(0, size)])
            handles = [
                pltpu.async_copy(
                    buf.at[pl.ds(0, size)],
                    out_hbm.at[b, pl.ds(r, size)], wsem)
                for b in range(batch)
            ]
            for h in handles:
                h.wait()

    return sc_copy(embed_table[:seq_len])


# speedup vs baseline: 1.0086x; 1.0086x over previous
import functools

import jax
import jax.numpy as jnp
from jax import lax
from jax.experimental import pallas as pl
from jax.experimental.pallas import tpu as pltpu
from jax.experimental.pallas import tpu_sc as plsc

NUM_CORES = 2
NUM_SUBCORES = 16
NUM_WORKERS = NUM_CORES * NUM_SUBCORES
CHUNK_ROWS = 88


def kernel(tokens, embed_table):
    batch = tokens.shape[0]
    seq_len = tokens.shape[1]
    embed_dim = embed_table.shape[1]
    rows_per_worker = seq_len // NUM_WORKERS
    chunks = []
    off = 0
    while off < rows_per_worker:
        size = min(CHUNK_ROWS, rows_per_worker - off)
        chunks.append((off, size))
        off += size
    mesh = plsc.VectorSubcoreMesh(core_axis_name="c", subcore_axis_name="s")

    @functools.partial(
        pl.kernel,
        mesh=mesh,
        out_type=jax.ShapeDtypeStruct(
            (batch, seq_len, embed_dim), embed_table.dtype),
        scratch_types=[
            pltpu.VMEM((CHUNK_ROWS, embed_dim), jnp.float32),
            pltpu.SemaphoreType.DMA,
        ],
    )
    def sc_copy(table_hbm, out_hbm, buf, wsem):
        wid = lax.axis_index("s") * NUM_CORES + lax.axis_index("c")
        base = wid * rows_per_worker

        for off, size in chunks:
            r = base + off
            pltpu.sync_copy(table_hbm.at[pl.ds(r, size)],
                            buf.at[pl.ds(0, size)])
            handles = [
                pltpu.async_copy(
                    buf.at[pl.ds(0, size)],
                    out_hbm.at[b, pl.ds(r, size)], wsem)
                for b in range(batch)
            ]
            for h in handles:
                h.wait()

    return sc_copy(embed_table[:seq_len])
